# Initial kernel scaffold; baseline (speedup 1.0000x reference)
import jax
import jax.numpy as jnp
from jax.experimental import pallas as pl


def kernel(point_features, point_cls_scores, point_coords, batch_size, W, b):
    raise NotImplementedError("write your pallas kernel here")


# speedup vs baseline: 1.4588x; 1.4588x over previous
import functools

import jax
import jax.numpy as jnp
from jax import lax
from jax.experimental import pallas as pl
from jax.experimental.pallas import tpu as pltpu
from jax.experimental.pallas import tpu_sc as plsc

_N = 32768
_D = 256
_B = 16
_NW = 32
_PPW = _N // _NW
_C = 128
_NCHUNK = _PPW // _C
_AW = 272
_ACC = _B * _AW


def _sc_partials(feats_flat, scores, coords_flat):
    mesh = plsc.VectorSubcoreMesh(
        core_axis_name="c", subcore_axis_name="s", num_cores=2, num_subcores=16
    )

    @functools.partial(
        pl.kernel,
        out_type=jax.ShapeDtypeStruct((_NW, _ACC), jnp.float32),
        mesh=mesh,
        scratch_types=[
            pltpu.VMEM((_C * _D,), jnp.float32),
            pltpu.VMEM((_PPW * 4,), jnp.int32),
            pltpu.VMEM((_PPW,), jnp.float32),
            pltpu.VMEM((_ACC,), jnp.float32),
        ],
    )
    def k(feat_hbm, scores_hbm, coords_hbm, out_hbm, fbuf, cbuf, sbuf, acc):
        w = lax.axis_index("s") * 2 + lax.axis_index("c")
        base = w * _PPW

        zero16 = jnp.zeros((16,), jnp.float32)

        def zbody(i, carry):
            acc[pl.ds(i * 16, 16)] = zero16
            return carry

        lax.fori_loop(0, _ACC // 16, zbody, 0)

        iota16 = lax.iota(jnp.int32, 16)
        e0 = (1 - jnp.minimum(iota16, 1)).astype(jnp.float32)

        pltpu.sync_copy(scores_hbm.at[pl.ds(base, _PPW)], sbuf)
        pltpu.sync_copy(coords_hbm.at[pl.ds(base * 4, _PPW * 4)], cbuf)


        def chunk(ci, carry):
            pltpu.sync_copy(
                feat_hbm.at[pl.ds((base + ci * _C) * _D, _C * _D)], fbuf
            )

            def grp(g, c2):
                p0 = ci * _C + g * 16
                svec = sbuf[pl.ds(p0, 16)]
                cvecs = [cbuf[pl.ds(p0 * 4 + q * 16, 16)] for q in range(4)]
                for i in range(16):
                    seg = cvecs[i // 4][(i % 4) * 4]
                    off = (seg & (_B - 1)) * _AW
                    sv = jnp.full((16,), svec[i], jnp.float32)
                    fb = (g * 16 + i) * _D
                    for j in range(_D // 16):
                        v = fbuf[pl.ds(fb + j * 16, 16)]
                        plsc.addupdate(acc.at[pl.ds(off + j * 16, 16)], sv * v)
                    plsc.addupdate(acc.at[pl.ds(off + 256, 16)], e0)
                return c2

            lax.fori_loop(0, _C // 16, grp, 0)
            return carry

        lax.fori_loop(0, _NCHUNK, chunk, 0)
        pltpu.sync_copy(acc, out_hbm.at[w])

    return k(feats_flat, scores, coords_flat)


def _head(partials3, W, b2):
    def k(p_ref, w_ref, b_ref, o_ref):
        x = p_ref[...]
        tot = jnp.sum(x, axis=0)
        sums = tot[:, :_D]
        cnt = jnp.sum(tot[:, _D:], axis=1)
        scene = sums / jnp.maximum(cnt, 1.0)[:, None]
        z = jnp.sum(scene * w_ref[...], axis=1, keepdims=True)
        o_ref[...] = jax.nn.sigmoid(z + b_ref[...])

    return pl.pallas_call(
        k,
        out_shape=jax.ShapeDtypeStruct((_B, 1), jnp.float32),
    )(partials3, W, b2)


def kernel(point_features, point_cls_scores, point_coords, batch_size, W, b):
    del batch_size
    feats_flat = point_features.reshape(-1)
    coords_flat = point_coords.reshape(-1)
    partials = _sc_partials(feats_flat, point_cls_scores, coords_flat)
    return _head(partials.reshape(_NW, _B, _AW), W, b.reshape(1, 1))


# speedup vs baseline: 2.2605x; 1.5496x over previous
import functools

import jax
import jax.numpy as jnp
from jax import lax
from jax.experimental import pallas as pl
from jax.experimental.pallas import tpu as pltpu
from jax.experimental.pallas import tpu_sc as plsc

_N = 32768
_D = 256
_B = 16
_NW = 32
_PPW = _N // _NW
_C = 128
_NCHUNK = _PPW // _C
_AW = 272
_ACC = _B * _AW


def _sc_partials(feats_flat, scores, coords_flat):
    mesh = plsc.VectorSubcoreMesh(
        core_axis_name="c", subcore_axis_name="s", num_cores=2, num_subcores=16
    )

    @functools.partial(
        pl.kernel,
        out_type=jax.ShapeDtypeStruct((_NW, _ACC), jnp.float32),
        mesh=mesh,
        scratch_types=[
            pltpu.VMEM((_C * _D,), jnp.float32),
            pltpu.VMEM((_PPW * 4,), jnp.int32),
            pltpu.VMEM((_PPW,), jnp.float32),
            pltpu.VMEM((_ACC,), jnp.float32),
        ],
    )
    def k(feat_hbm, scores_hbm, coords_hbm, out_hbm, fbuf, cbuf, sbuf, acc):
        w = lax.axis_index("s") * 2 + lax.axis_index("c")
        base = w * _PPW

        zero16 = jnp.zeros((16,), jnp.float32)

        def zbody(i, carry):
            acc[pl.ds(i * 16, 16)] = zero16
            return carry

        lax.fori_loop(0, _ACC // 16, zbody, 0)

        iota16 = lax.iota(jnp.int32, 16)
        e0 = (1 - jnp.minimum(iota16, 1)).astype(jnp.float32)

        pltpu.sync_copy(scores_hbm.at[pl.ds(base, _PPW)], sbuf)
        pltpu.sync_copy(coords_hbm.at[pl.ds(base * 4, _PPW * 4)], cbuf)


        def chunk(ci, carry):
            pltpu.sync_copy(
                feat_hbm.at[pl.ds((base + ci * _C) * _D, _C * _D)], fbuf
            )

            def grp(g, c2):
                p0 = ci * _C + g * 16
                svec = sbuf[pl.ds(p0, 16)]
                cvecs = [cbuf[pl.ds(p0 * 4 + q * 16, 16)] for q in range(4)]
                for i in range(16):
                    seg = cvecs[i // 4][(i % 4) * 4]
                    off = (seg & (_B - 1)) * _AW
                    sv = jnp.full((16,), svec[i], jnp.float32)
                    fb = (g * 16 + i) * _D
                    vs = [fbuf[pl.ds(fb + j * 16, 16)] for j in range(_D // 16)]
                    ps = [sv * v for v in vs]
                    for j in range(_D // 16):
                        plsc.addupdate(acc.at[pl.ds(off + j * 16, 16)], ps[j])
                    plsc.addupdate(acc.at[pl.ds(off + 256, 16)], e0)
                return c2

            lax.fori_loop(0, _C // 16, grp, 0)
            return carry

        lax.fori_loop(0, _NCHUNK, chunk, 0)
        pltpu.sync_copy(acc, out_hbm.at[w])

    return k(feats_flat, scores, coords_flat)


def _head(partials3, W, b2):
    def k(p_ref, w_ref, b_ref, o_ref):
        x = p_ref[...]
        tot = jnp.sum(x, axis=0)
        sums = tot[:, :_D]
        cnt = jnp.sum(tot[:, _D:], axis=1)
        scene = sums / jnp.maximum(cnt, 1.0)[:, None]
        z = jnp.sum(scene * w_ref[...], axis=1, keepdims=True)
        o_ref[...] = jax.nn.sigmoid(z + b_ref[...])

    return pl.pallas_call(
        k,
        out_shape=jax.ShapeDtypeStruct((_B, 1), jnp.float32),
    )(partials3, W, b2)


def kernel(point_features, point_cls_scores, point_coords, batch_size, W, b):
    del batch_size
    feats_flat = point_features.reshape(-1)
    coords_flat = point_coords.reshape(-1)
    partials = _sc_partials(feats_flat, point_cls_scores, coords_flat)
    return _head(partials.reshape(_NW, _B, _AW), W, b.reshape(1, 1))


# speedup vs baseline: 2.6854x; 1.1879x over previous
import functools

import jax
import jax.numpy as jnp
from jax import lax
from jax.experimental import pallas as pl
from jax.experimental.pallas import tpu as pltpu
from jax.experimental.pallas import tpu_sc as plsc

_N = 32768
_D = 256
_B = 16
_NW = 32
_PPW = _N // _NW
_C = 128
_NCHUNK = _PPW // _C
_AW = 272
_ACC = _B * _AW


def _sc_partials(feats, scores, coords_flat):
    mesh = plsc.VectorSubcoreMesh(
        core_axis_name="c", subcore_axis_name="s", num_cores=2, num_subcores=16
    )

    @functools.partial(
        pl.kernel,
        out_type=jax.ShapeDtypeStruct((_NW, _ACC), jnp.float32),
        mesh=mesh,
        scratch_types=[
            pltpu.VMEM((_C, _D), jnp.float32),
            pltpu.VMEM((_PPW * 4,), jnp.int32),
            pltpu.VMEM((_PPW,), jnp.float32),
            pltpu.VMEM((_ACC,), jnp.float32),
        ],
        compiler_params=pltpu.CompilerParams(use_tc_tiling_on_sc=True),
    )
    def k(feat_hbm, scores_hbm, coords_hbm, out_hbm, fbuf, cbuf, sbuf, acc):
        w = lax.axis_index("s") * 2 + lax.axis_index("c")
        base = w * _PPW

        zero16 = jnp.zeros((16,), jnp.float32)

        def zbody(i, carry):
            acc[pl.ds(i * 16, 16)] = zero16
            return carry

        lax.fori_loop(0, _ACC // 16, zbody, 0)

        iota16 = lax.iota(jnp.int32, 16)
        e0 = (1 - jnp.minimum(iota16, 1)).astype(jnp.float32)

        pltpu.sync_copy(scores_hbm.at[pl.ds(base, _PPW)], sbuf)
        pltpu.sync_copy(coords_hbm.at[pl.ds(base * 4, _PPW * 4)], cbuf)


        def chunk(ci, carry):
            pltpu.sync_copy(
                feat_hbm.at[pl.ds(base + ci * _C, _C), :], fbuf
            )

            def grp(g, c2):
                p0 = ci * _C + g * 16
                svec = sbuf[pl.ds(p0, 16)]
                cvecs = [cbuf[pl.ds(p0 * 4 + q * 16, 16)] for q in range(4)]
                for i in range(16):
                    seg = cvecs[i // 4][(i % 4) * 4]
                    off = (seg & (_B - 1)) * _AW
                    sv = jnp.full((16,), svec[i], jnp.float32)
                    r = g * 16 + i
                    vs = [fbuf[r, pl.ds(j * 16, 16)] for j in range(_D // 16)]
                    ps = [sv * v for v in vs]
                    for j in range(_D // 16):
                        plsc.addupdate(acc.at[pl.ds(off + j * 16, 16)], ps[j])
                    plsc.addupdate(acc.at[pl.ds(off + 256, 16)], e0)
                return c2

            lax.fori_loop(0, _C // 16, grp, 0)
            return carry

        lax.fori_loop(0, _NCHUNK, chunk, 0)
        pltpu.sync_copy(acc, out_hbm.at[w])

    return k(feats, scores, coords_flat)


def _head(partials3, W, b2):
    def k(p_ref, w_ref, b_ref, o_ref):
        x = p_ref[...]
        tot = jnp.sum(x, axis=0)
        sums = tot[:, :_D]
        cnt = jnp.sum(tot[:, _D:], axis=1)
        scene = sums / jnp.maximum(cnt, 1.0)[:, None]
        z = jnp.sum(scene * w_ref[...], axis=1, keepdims=True)
        o_ref[...] = jax.nn.sigmoid(z + b_ref[...])

    return pl.pallas_call(
        k,
        out_shape=jax.ShapeDtypeStruct((_B, 1), jnp.float32),
    )(partials3, W, b2)


def kernel(point_features, point_cls_scores, point_coords, batch_size, W, b):
    del batch_size
    coords_flat = point_coords.reshape(-1)
    partials = _sc_partials(point_features, point_cls_scores, coords_flat)
    return _head(partials.reshape(_NW, _B, _AW), W, b.reshape(1, 1))


# speedup vs baseline: 4.0560x; 1.5104x over previous
import functools

import jax
import jax.numpy as jnp
from jax import lax
from jax.experimental import pallas as pl
from jax.experimental.pallas import tpu as pltpu
from jax.experimental.pallas import tpu_sc as plsc

_N = 32768
_D = 256
_B = 16
_NW = 32
_PPW = _N // _NW
_C = 128
_NCHUNK = _PPW // _C
_AW = 272
_ACC = _B * _AW


def _sc_partials(feats, scores, segs):
    mesh = plsc.VectorSubcoreMesh(
        core_axis_name="c", subcore_axis_name="s", num_cores=2, num_subcores=16
    )

    @functools.partial(
        pl.kernel,
        out_type=jax.ShapeDtypeStruct((_NW, _ACC), jnp.float32),
        mesh=mesh,
        scratch_types=[
            pltpu.VMEM((_C, _D), jnp.float32),
            pltpu.VMEM((_C, _D), jnp.float32),
            pltpu.VMEM((_PPW,), jnp.int32),
            pltpu.VMEM((_PPW,), jnp.float32),
            pltpu.VMEM((_ACC,), jnp.float32),
            pltpu.SemaphoreType.DMA,
            pltpu.SemaphoreType.DMA,
        ],
        compiler_params=pltpu.CompilerParams(use_tc_tiling_on_sc=True),
    )
    def k(feat_hbm, scores_hbm, segs_hbm, out_hbm,
          fbuf0, fbuf1, cbuf, sbuf, acc, sem0, sem1):
        w = lax.axis_index("s") * 2 + lax.axis_index("c")
        base = w * _PPW

        bufs = (fbuf0, fbuf1)
        sems = (sem0, sem1)

        def feat_copy(ci):
            return pltpu.make_async_copy(
                feat_hbm.at[pl.ds(base + ci * _C, _C), :],
                bufs[ci % 2],
                sems[ci % 2],
            )

        feat_copy(0).start()

        zero16 = jnp.zeros((16,), jnp.float32)

        def zbody(i, carry):
            acc[pl.ds(i * 16, 16)] = zero16
            return carry

        lax.fori_loop(0, _ACC // 16, zbody, 0)

        iota16 = lax.iota(jnp.int32, 16)
        e0 = (1 - jnp.minimum(iota16, 1)).astype(jnp.float32)

        pltpu.sync_copy(scores_hbm.at[pl.ds(base, _PPW)], sbuf)
        pltpu.sync_copy(segs_hbm.at[pl.ds(base, _PPW)], cbuf)

        for ci in range(_NCHUNK):
            if ci + 1 < _NCHUNK:
                feat_copy(ci + 1).start()
            feat_copy(ci).wait()
            fbuf = bufs[ci % 2]

            def grp(g, c2, ci=ci, fbuf=fbuf):
                p0 = ci * _C + g * 16
                svec = sbuf[pl.ds(p0, 16)]
                cvec = cbuf[pl.ds(p0, 16)]
                for i in range(16):
                    off = (cvec[i] & (_B - 1)) * _AW
                    sv = jnp.full((16,), svec[i], jnp.float32)
                    r = g * 16 + i
                    vs = [fbuf[r, pl.ds(j * 16, 16)] for j in range(_D // 16)]
                    ps = [sv * v for v in vs]
                    for j in range(_D // 16):
                        plsc.addupdate(acc.at[pl.ds(off + j * 16, 16)], ps[j])
                    plsc.addupdate(acc.at[pl.ds(off + 256, 16)], e0)
                return c2

            lax.fori_loop(0, _C // 16, grp, 0)

        pltpu.sync_copy(acc, out_hbm.at[w])

    return k(feats, scores, segs)


def _head(partials3, W, b2):
    def k(p_ref, w_ref, b_ref, o_ref):
        x = p_ref[...]
        tot = jnp.sum(x, axis=0)
        sums = tot[:, :_D]
        cnt = jnp.sum(tot[:, _D:], axis=1)
        scene = sums / jnp.maximum(cnt, 1.0)[:, None]
        z = jnp.sum(scene * w_ref[...], axis=1, keepdims=True)
        o_ref[...] = jax.nn.sigmoid(z + b_ref[...])

    return pl.pallas_call(
        k,
        out_shape=jax.ShapeDtypeStruct((_B, 1), jnp.float32),
    )(partials3, W, b2)


def kernel(point_features, point_cls_scores, point_coords, batch_size, W, b):
    del batch_size
    segs = point_coords[:, 0]
    partials = _sc_partials(point_features, point_cls_scores, segs)
    return _head(partials.reshape(_NW, _B, _AW), W, b.reshape(1, 1))


# speedup vs baseline: 4.0743x; 1.0045x over previous
import functools

import jax
import jax.numpy as jnp
from jax import lax
from jax.experimental import pallas as pl
from jax.experimental.pallas import tpu as pltpu
from jax.experimental.pallas import tpu_sc as plsc

_N = 32768
_D = 256
_B = 16
_NW = 32
_PPW = _N // _NW
_C = 128
_NCHUNK = _PPW // _C
_AW = 272
_ACC = _B * _AW


def _sc_partials(feats, scores, segs):
    mesh = plsc.VectorSubcoreMesh(
        core_axis_name="c", subcore_axis_name="s", num_cores=2, num_subcores=16
    )

    @functools.partial(
        pl.kernel,
        out_type=jax.ShapeDtypeStruct((_NW, _ACC), jnp.float32),
        mesh=mesh,
        scratch_types=[
            pltpu.VMEM((_C, _D), jnp.float32),
            pltpu.VMEM((_C, _D), jnp.float32),
            pltpu.VMEM((_PPW,), jnp.int32),
            pltpu.VMEM((_PPW,), jnp.float32),
            pltpu.VMEM((_ACC,), jnp.float32),
            pltpu.SemaphoreType.DMA,
            pltpu.SemaphoreType.DMA,
        ],
        compiler_params=pltpu.CompilerParams(use_tc_tiling_on_sc=True),
    )
    def k(feat_hbm, scores_hbm, segs_hbm, out_hbm,
          fbuf0, fbuf1, cbuf, sbuf, acc, sem0, sem1):
        w = lax.axis_index("s") * 2 + lax.axis_index("c")
        base = w * _PPW

        bufs = (fbuf0, fbuf1)
        sems = (sem0, sem1)

        def feat_copy(ci):
            return pltpu.make_async_copy(
                feat_hbm.at[pl.ds(base + ci * _C, _C), :],
                bufs[ci % 2],
                sems[ci % 2],
            )

        feat_copy(0).start()

        zero16 = jnp.zeros((16,), jnp.float32)

        def zbody(i, carry):
            acc[pl.ds(i * 16, 16)] = zero16
            return carry

        lax.fori_loop(0, _ACC // 16, zbody, 0)

        iota16 = lax.iota(jnp.int32, 16)
        e0 = (1 - jnp.minimum(iota16, 1)).astype(jnp.float32)

        pltpu.sync_copy(scores_hbm.at[pl.ds(base, _PPW)], sbuf)
        pltpu.sync_copy(segs_hbm.at[pl.ds(base, _PPW)], cbuf)

        for ci in range(_NCHUNK):
            if ci + 1 < _NCHUNK:
                feat_copy(ci + 1).start()
            feat_copy(ci).wait()
            fbuf = bufs[ci % 2]

            def grp(g, c2, ci=ci, fbuf=fbuf):
                p0 = ci * _C + g * 16
                svec = sbuf[pl.ds(p0, 16)]
                cvec = cbuf[pl.ds(p0, 16)]

                def point_loads(i):
                    off = (cvec[i] & (_B - 1)) * _AW
                    sv = jnp.full((16,), svec[i], jnp.float32)
                    r = g * 16 + i
                    vs = [fbuf[r, pl.ds(j * 16, 16)] for j in range(_D // 16)]
                    return off, sv, vs

                def point_store(st):
                    off, sv, vs = st
                    ps = [sv * v for v in vs]
                    for j in range(_D // 16):
                        plsc.addupdate(acc.at[pl.ds(off + j * 16, 16)], ps[j])
                    plsc.addupdate(acc.at[pl.ds(off + 256, 16)], e0)

                prev = point_loads(0)
                for i in range(1, 16):
                    cur = point_loads(i)
                    point_store(prev)
                    prev = cur
                point_store(prev)
                return c2

            lax.fori_loop(0, _C // 16, grp, 0)

        pltpu.sync_copy(acc, out_hbm.at[w])

    return k(feats, scores, segs)


def _head(partials3, W, b2):
    def k(p_ref, w_ref, b_ref, o_ref):
        x = p_ref[...]
        tot = jnp.sum(x, axis=0)
        sums = tot[:, :_D]
        cnt = jnp.sum(tot[:, _D:], axis=1)
        scene = sums / jnp.maximum(cnt, 1.0)[:, None]
        z = jnp.sum(scene * w_ref[...], axis=1, keepdims=True)
        o_ref[...] = jax.nn.sigmoid(z + b_ref[...])

    return pl.pallas_call(
        k,
        out_shape=jax.ShapeDtypeStruct((_B, 1), jnp.float32),
    )(partials3, W, b2)


def kernel(point_features, point_cls_scores, point_coords, batch_size, W, b):
    del batch_size
    segs = point_coords[:, 0]
    partials = _sc_partials(point_features, point_cls_scores, segs)
    return _head(partials.reshape(_NW, _B, _AW), W, b.reshape(1, 1))


# speedup vs baseline: 4.2644x; 1.0466x over previous
import functools

import jax
import jax.numpy as jnp
from jax import lax
from jax.experimental import pallas as pl
from jax.experimental.pallas import tpu as pltpu
from jax.experimental.pallas import tpu_sc as plsc

_N = 32768
_D = 256
_B = 16
_NW = 32
_PPW = _N // _NW
_C = 128
_NCHUNK = _PPW // _C
_AW = 272
_ACC = _B * _AW


def _sc_partials(feats, scores, segs):
    mesh = plsc.VectorSubcoreMesh(
        core_axis_name="c", subcore_axis_name="s", num_cores=2, num_subcores=16
    )

    @functools.partial(
        pl.kernel,
        out_type=jax.ShapeDtypeStruct((_NW, _B, _AW), jnp.float32),
        mesh=mesh,
        scratch_types=[
            pltpu.VMEM((_C, _D), jnp.float32),
            pltpu.VMEM((_C, _D), jnp.float32),
            pltpu.VMEM((_PPW,), jnp.int32),
            pltpu.VMEM((_PPW,), jnp.float32),
            pltpu.VMEM((_B, _AW), jnp.float32),
            pltpu.SemaphoreType.DMA,
            pltpu.SemaphoreType.DMA,
        ],
        compiler_params=pltpu.CompilerParams(use_tc_tiling_on_sc=True),
    )
    def k(feat_hbm, scores_hbm, segs_hbm, out_hbm,
          fbuf0, fbuf1, cbuf, sbuf, acc, sem0, sem1):
        w = lax.axis_index("s") * 2 + lax.axis_index("c")
        base = w * _PPW

        bufs = (fbuf0, fbuf1)
        sems = (sem0, sem1)

        def feat_copy(ci):
            return pltpu.make_async_copy(
                feat_hbm.at[pl.ds(base + ci * _C, _C), :],
                bufs[ci % 2],
                sems[ci % 2],
            )

        feat_copy(0).start()

        zero16 = jnp.zeros((16,), jnp.float32)

        def zbody(r, carry):
            for kk in range(_AW // 16):
                acc[r, pl.ds(kk * 16, 16)] = zero16
            return carry

        lax.fori_loop(0, _B, zbody, 0)

        iota16 = lax.iota(jnp.int32, 16)
        e0 = (1 - jnp.minimum(iota16, 1)).astype(jnp.float32)

        pltpu.sync_copy(scores_hbm.at[pl.ds(base, _PPW)], sbuf)
        pltpu.sync_copy(segs_hbm.at[pl.ds(base, _PPW)], cbuf)

        for ci in range(_NCHUNK):
            if ci + 1 < _NCHUNK:
                feat_copy(ci + 1).start()
            feat_copy(ci).wait()
            fbuf = bufs[ci % 2]

            def grp(g, c2, ci=ci, fbuf=fbuf):
                p0 = ci * _C + g * 16
                svec = sbuf[pl.ds(p0, 16)]
                cvec = cbuf[pl.ds(p0, 16)]

                def point_loads(i):
                    seg = cvec[i] & (_B - 1)
                    sv = jnp.full((16,), svec[i], jnp.float32)
                    r = g * 16 + i
                    vs = [fbuf[r, pl.ds(j * 16, 16)] for j in range(_D // 16)]
                    return seg, sv, vs

                def point_store(st):
                    seg, sv, vs = st
                    ps = [sv * v for v in vs]
                    for j in range(_D // 16):
                        plsc.addupdate(acc.at[seg, pl.ds(j * 16, 16)], ps[j])
                    plsc.addupdate(acc.at[seg, pl.ds(256, 16)], e0)

                prev = point_loads(0)
                for i in range(1, 16):
                    cur = point_loads(i)
                    point_store(prev)
                    prev = cur
                point_store(prev)
                return c2

            lax.fori_loop(0, _C // 16, grp, 0)

        pltpu.sync_copy(acc, out_hbm.at[w])

    return k(feats, scores, segs)


def _head(partials3, W, b2):
    def k(p_ref, w_ref, b_ref, o_ref):
        x = p_ref[...]
        tot = jnp.sum(x, axis=0)
        sums = tot[:, :_D]
        cnt = jnp.sum(tot[:, _D:], axis=1)
        scene = sums / jnp.maximum(cnt, 1.0)[:, None]
        z = jnp.sum(scene * w_ref[...], axis=1, keepdims=True)
        o_ref[...] = jax.nn.sigmoid(z + b_ref[...])

    return pl.pallas_call(
        k,
        out_shape=jax.ShapeDtypeStruct((_B, 1), jnp.float32),
    )(partials3, W, b2)


def kernel(point_features, point_cls_scores, point_coords, batch_size, W, b):
    del batch_size
    segs = point_coords[:, 0]
    partials = _sc_partials(point_features, point_cls_scores, segs)
    return _head(partials, W, b.reshape(1, 1))
